# Initial kernel scaffold; baseline (speedup 1.0000x reference)
import jax
import jax.numpy as jnp
from jax.experimental import pallas as pl


def kernel(x, edge_index, W1, att_src1, att_dst1, b1, W2, att_src2, att_dst2, b2, lin_W, lin_b):
    raise NotImplementedError("write your pallas kernel here")


# speedup vs baseline: 19.0279x; 19.0279x over previous
import functools

import jax
import jax.numpy as jnp
import numpy as np
from jax import lax
from jax.experimental import pallas as pl
from jax.experimental.pallas import tpu as pltpu, tpu_sc as plsc

N = 10000
NP = 10240
E = 320000
IN_DIM = 128
HID = 32
HEADS = 4
F1 = HEADS * HID
F2 = HID
OUT = 2
BN = 2048
CH = 1600


def _lrelu(x):
    return jnp.maximum(x, 0.2 * x)


def _elu(x):
    return jnp.where(x > 0, x, jnp.exp(jnp.minimum(x, 0.0)) - 1.0)


def _tc_pre1_body(x_ref, w1_ref, asrc_ref, adst_ref, sel_ref,
                  h1t_ref, as_ref, ad_ref, num0_ref, den0_ref):
    xb = x_ref[...]
    h1t = lax.dot_general(w1_ref[...], xb, (((0,), (1,)), ((), ())),
                          preferred_element_type=jnp.float32)
    a_s = lax.dot_general(asrc_ref[...], h1t, (((1,), (0,)), ((), ())),
                          preferred_element_type=jnp.float32)
    a_d = lax.dot_general(adst_ref[...], h1t, (((1,), (0,)), ((), ())),
                          preferred_element_type=jnp.float32)
    p_self = jnp.exp(_lrelu(a_s + a_d))
    p_full = lax.dot_general(sel_ref[...], p_self, (((1,), (0,)), ((), ())),
                             preferred_element_type=jnp.float32)
    h1t_ref[...] = h1t
    as_ref[...] = a_s
    ad_ref[...] = a_d
    num0_ref[...] = h1t * p_full
    den0_ref[...] = p_self


def _tc_pre1(x_pad, W1, A_src, A_dst, Sel):
    grid = (NP // BN,)
    return pl.pallas_call(
        _tc_pre1_body,
        grid=grid,
        in_specs=[
            pl.BlockSpec((BN, IN_DIM), lambda i: (i, 0)),
            pl.BlockSpec((IN_DIM, F1), lambda i: (0, 0)),
            pl.BlockSpec((HEADS, F1), lambda i: (0, 0)),
            pl.BlockSpec((HEADS, F1), lambda i: (0, 0)),
            pl.BlockSpec((F1, HEADS), lambda i: (0, 0)),
        ],
        out_specs=[
            pl.BlockSpec((F1, BN), lambda i: (0, i)),
            pl.BlockSpec((HEADS, BN), lambda i: (0, i)),
            pl.BlockSpec((HEADS, BN), lambda i: (0, i)),
            pl.BlockSpec((F1, BN), lambda i: (0, i)),
            pl.BlockSpec((HEADS, BN), lambda i: (0, i)),
        ],
        out_shape=[
            jax.ShapeDtypeStruct((F1, NP), jnp.float32),
            jax.ShapeDtypeStruct((HEADS, NP), jnp.float32),
            jax.ShapeDtypeStruct((HEADS, NP), jnp.float32),
            jax.ShapeDtypeStruct((F1, NP), jnp.float32),
            jax.ShapeDtypeStruct((HEADS, NP), jnp.float32),
        ],
    )(x_pad, W1, A_src, A_dst, Sel)


_SC_PARAMS = pltpu.CompilerParams(
    needs_layout_passes=False, use_tc_tiling_on_sc=False
)
_MESH = dict(core_axis_name="c", subcore_axis_name="s")


def _sc_edge1(src, dst, a_srcT, a_dstT, h1T, num_init, den_init):
    mesh = plsc.VectorSubcoreMesh(**_MESH)

    @functools.partial(
        pl.kernel,
        out_type=[
            jax.ShapeDtypeStruct((F1, NP), jnp.float32),
            jax.ShapeDtypeStruct((HEADS, NP), jnp.float32),
        ],
        mesh=mesh,
        scratch_types=[
            pltpu.VMEM((4, NP), jnp.float32),
            pltpu.VMEM((4, NP), jnp.float32),
            pltpu.VMEM((1, NP), jnp.float32),
            pltpu.VMEM((1, NP), jnp.float32),
            pltpu.VMEM((1, NP), jnp.float32),
            pltpu.VMEM((CH,), jnp.int32),
            pltpu.VMEM((CH,), jnp.int32),
        ],
        compiler_params=_SC_PARAMS,
    )
    def k(src_h, dst_h, as_h, ad_h, h1_h, n0_h, d0_h, num_o, den_o,
          hc, acc, asr, adr, dacc, sbuf, dbuf):
        t = lax.axis_index("s") * 2 + lax.axis_index("c")
        head = t // 8
        pltpu.sync_copy(h1_h.at[pl.ds(4 * t, 4)], hc)
        pltpu.sync_copy(n0_h.at[pl.ds(4 * t, 4)], acc)
        pltpu.sync_copy(as_h.at[pl.ds(head, 1)], asr)
        pltpu.sync_copy(ad_h.at[pl.ds(head, 1)], adr)
        pltpu.sync_copy(d0_h.at[pl.ds(head, 1)], dacc)

        f0 = jnp.full((16,), 0, jnp.int32)
        f1 = jnp.full((16,), 1, jnp.int32)
        f2 = jnp.full((16,), 2, jnp.int32)
        f3 = jnp.full((16,), 3, jnp.int32)

        def chunk_body(g, _):
            pltpu.sync_copy(src_h.at[pl.ds(g * CH, CH)], sbuf)
            pltpu.sync_copy(dst_h.at[pl.ds(g * CH, CH)], dbuf)

            def blk(i, _):
                sv = sbuf[pl.ds(i * 16, 16)]
                dv = dbuf[pl.ds(i * 16, 16)]
                e = plsc.load_gather(asr, [f0, sv]) + plsc.load_gather(adr, [f0, dv])
                p = jnp.exp(jnp.maximum(e, 0.2 * e))
                plsc.addupdate_scatter(dacc, [f0, dv], p)
                for fv in (f0, f1, f2, f3):
                    hv = plsc.load_gather(hc, [fv, sv])
                    plsc.addupdate_scatter(acc, [fv, dv], hv * p)
                return 0

            lax.fori_loop(0, CH // 16, blk, 0)
            return 0

        lax.fori_loop(0, E // CH, chunk_body, 0)

        pltpu.sync_copy(acc, num_o.at[pl.ds(4 * t, 4)])

        @pl.when(t % 8 == 0)
        def _():
            pltpu.sync_copy(dacc, den_o.at[pl.ds(head, 1)])

    return k(src, dst, a_srcT, a_dstT, h1T, num_init, den_init)


def _tc_mid_body(num_ref, den_ref, b1_ref, w2_ref, as2_ref, ad2_ref, sel_ref,
                 h2t_ref, as_ref, ad_ref, num0_ref, den0_ref):
    den_full = lax.dot_general(sel_ref[...], den_ref[...] + 1e-16,
                               (((1,), (0,)), ((), ())),
                               preferred_element_type=jnp.float32)
    out1 = num_ref[...] / den_full + b1_ref[...]
    out1 = _elu(out1)
    h2t = lax.dot_general(w2_ref[...], out1, (((0,), (0,)), ((), ())),
                          preferred_element_type=jnp.float32)
    a_s = lax.dot_general(as2_ref[...], h2t, (((1,), (0,)), ((), ())),
                          preferred_element_type=jnp.float32)
    a_d = lax.dot_general(ad2_ref[...], h2t, (((1,), (0,)), ((), ())),
                          preferred_element_type=jnp.float32)
    p_self = jnp.exp(_lrelu(a_s + a_d))
    h2t_ref[...] = h2t
    as_ref[...] = a_s
    ad_ref[...] = a_d
    num0_ref[...] = h2t * p_self
    den0_ref[...] = p_self


def _tc_mid(num1, den1, b1c, W2, a2s, a2d, Sel):
    grid = (NP // BN,)
    return pl.pallas_call(
        _tc_mid_body,
        grid=grid,
        in_specs=[
            pl.BlockSpec((F1, BN), lambda i: (0, i)),
            pl.BlockSpec((HEADS, BN), lambda i: (0, i)),
            pl.BlockSpec((F1, 1), lambda i: (0, 0)),
            pl.BlockSpec((F1, F2), lambda i: (0, 0)),
            pl.BlockSpec((1, F2), lambda i: (0, 0)),
            pl.BlockSpec((1, F2), lambda i: (0, 0)),
            pl.BlockSpec((F1, HEADS), lambda i: (0, 0)),
        ],
        out_specs=[
            pl.BlockSpec((F2, BN), lambda i: (0, i)),
            pl.BlockSpec((1, BN), lambda i: (0, i)),
            pl.BlockSpec((1, BN), lambda i: (0, i)),
            pl.BlockSpec((F2, BN), lambda i: (0, i)),
            pl.BlockSpec((1, BN), lambda i: (0, i)),
        ],
        out_shape=[
            jax.ShapeDtypeStruct((F2, NP), jnp.float32),
            jax.ShapeDtypeStruct((1, NP), jnp.float32),
            jax.ShapeDtypeStruct((1, NP), jnp.float32),
            jax.ShapeDtypeStruct((F2, NP), jnp.float32),
            jax.ShapeDtypeStruct((1, NP), jnp.float32),
        ],
    )(num1, den1, b1c, W2, a2s, a2d, Sel)


def _sc_edge2(src, dst, a_src2, a_dst2, h2T, num_init, den_init):
    mesh = plsc.VectorSubcoreMesh(**_MESH)

    @functools.partial(
        pl.kernel,
        out_type=[
            jax.ShapeDtypeStruct((F2, NP), jnp.float32),
            jax.ShapeDtypeStruct((1, NP), jnp.float32),
        ],
        mesh=mesh,
        scratch_types=[
            pltpu.VMEM((1, NP), jnp.float32),
            pltpu.VMEM((1, NP), jnp.float32),
            pltpu.VMEM((1, NP), jnp.float32),
            pltpu.VMEM((1, NP), jnp.float32),
            pltpu.VMEM((1, NP), jnp.float32),
            pltpu.VMEM((CH,), jnp.int32),
            pltpu.VMEM((CH,), jnp.int32),
        ],
        compiler_params=_SC_PARAMS,
    )
    def k(src_h, dst_h, as_h, ad_h, h2_h, n0_h, d0_h, num_o, den_o,
          hc, acc, asr, adr, dacc, sbuf, dbuf):
        t = lax.axis_index("s") * 2 + lax.axis_index("c")
        pltpu.sync_copy(h2_h.at[pl.ds(t, 1)], hc)
        pltpu.sync_copy(n0_h.at[pl.ds(t, 1)], acc)
        pltpu.sync_copy(as_h, asr)
        pltpu.sync_copy(ad_h, adr)
        pltpu.sync_copy(d0_h, dacc)

        f0 = jnp.full((16,), 0, jnp.int32)

        def chunk_body(g, _):
            pltpu.sync_copy(src_h.at[pl.ds(g * CH, CH)], sbuf)
            pltpu.sync_copy(dst_h.at[pl.ds(g * CH, CH)], dbuf)

            def blk(i, _):
                sv = sbuf[pl.ds(i * 16, 16)]
                dv = dbuf[pl.ds(i * 16, 16)]
                e = plsc.load_gather(asr, [f0, sv]) + plsc.load_gather(adr, [f0, dv])
                p = jnp.exp(jnp.maximum(e, 0.2 * e))
                plsc.addupdate_scatter(dacc, [f0, dv], p)
                hv = plsc.load_gather(hc, [f0, sv])
                plsc.addupdate_scatter(acc, [f0, dv], hv * p)
                return 0

            lax.fori_loop(0, CH // 16, blk, 0)
            return 0

        lax.fori_loop(0, E // CH, chunk_body, 0)

        pltpu.sync_copy(acc, num_o.at[pl.ds(t, 1)])

        @pl.when(t == 0)
        def _():
            pltpu.sync_copy(dacc, den_o)

    return k(src, dst, a_src2, a_dst2, h2T, num_init, den_init)


def _tc_post_body(num_ref, den_ref, b2_ref, lw_ref, lb_ref, out_ref):
    out2 = num_ref[...] / (den_ref[...] + 1e-16) + b2_ref[...]
    out2 = _elu(out2)
    y = lax.dot_general(lw_ref[...], out2, (((0,), (0,)), ((), ())),
                        preferred_element_type=jnp.float32)
    out_ref[...] = y + lb_ref[...]


def _tc_post(num2, den2, b2c, lin_W, lin_bc):
    grid = (NP // BN,)
    return pl.pallas_call(
        _tc_post_body,
        grid=grid,
        in_specs=[
            pl.BlockSpec((F2, BN), lambda i: (0, i)),
            pl.BlockSpec((1, BN), lambda i: (0, i)),
            pl.BlockSpec((F2, 1), lambda i: (0, 0)),
            pl.BlockSpec((F2, OUT), lambda i: (0, 0)),
            pl.BlockSpec((OUT, 1), lambda i: (0, 0)),
        ],
        out_specs=pl.BlockSpec((OUT, BN), lambda i: (0, i)),
        out_shape=jax.ShapeDtypeStruct((OUT, NP), jnp.float32),
    )(num2, den2, b2c, lin_W, lin_bc)


def kernel(x, edge_index, W1, att_src1, att_dst1, b1, W2, att_src2, att_dst2,
           b2, lin_W, lin_b):
    x_pad = jnp.pad(x, ((0, NP - N), (0, 0)))
    src = edge_index[0]
    dst = edge_index[1]

    head_of = np.arange(F1) // HID
    blockmask = jnp.asarray(
        (np.arange(HEADS)[:, None] == head_of[None, :]).astype(np.float32))
    A_src = blockmask * att_src1.reshape(1, F1)
    A_dst = blockmask * att_dst1.reshape(1, F1)
    Sel = blockmask.T

    h1T, a_sT, a_dT, num0, den0 = _tc_pre1(x_pad, W1, A_src, A_dst, Sel)
    num1, den1 = _sc_edge1(src, dst, a_sT, a_dT, h1T, num0, den0)

    h2T, a2sT, a2dT, num0b, den0b = _tc_mid(
        num1, den1, b1.reshape(F1, 1), W2,
        att_src2.reshape(1, F2), att_dst2.reshape(1, F2), Sel)
    num2, den2 = _sc_edge2(src, dst, a2sT, a2dT, h2T, num0b, den0b)

    yT = _tc_post(num2, den2, b2.reshape(F2, 1), lin_W,
                  lin_b.reshape(OUT, 1))
    return yT[:, :N].T


# speedup vs baseline: 25.0075x; 1.3143x over previous
import functools

import jax
import jax.numpy as jnp
import numpy as np
from jax import lax
from jax.experimental import pallas as pl
from jax.experimental.pallas import tpu as pltpu, tpu_sc as plsc

N = 10000
NP = 10240
E = 320000
IN_DIM = 128
HID = 32
HEADS = 4
F1 = HEADS * HID
F2 = HID
OUT = 2
BN = 2048
CH = 1600


def _lrelu(x):
    return jnp.maximum(x, 0.2 * x)


def _elu(x):
    return jnp.where(x > 0, x, jnp.exp(jnp.minimum(x, 0.0)) - 1.0)


def _tc_pre1_body(x_ref, w1_ref, asrc_ref, adst_ref, sel_ref,
                  h1t_ref, as_ref, ad_ref, num0_ref, den0_ref):
    xb = x_ref[...]
    h1t = lax.dot_general(w1_ref[...], xb, (((0,), (1,)), ((), ())),
                          preferred_element_type=jnp.float32)
    a_s = lax.dot_general(asrc_ref[...], h1t, (((1,), (0,)), ((), ())),
                          preferred_element_type=jnp.float32)
    a_d = lax.dot_general(adst_ref[...], h1t, (((1,), (0,)), ((), ())),
                          preferred_element_type=jnp.float32)
    p_self = jnp.exp(_lrelu(a_s + a_d))
    p_full = lax.dot_general(sel_ref[...], p_self, (((1,), (0,)), ((), ())),
                             preferred_element_type=jnp.float32)
    h1t_ref[...] = h1t
    as_ref[...] = a_s
    ad_ref[...] = a_d
    num0_ref[...] = h1t * p_full
    den0_ref[...] = p_self


def _tc_pre1(x_pad, W1, A_src, A_dst, Sel):
    grid = (NP // BN,)
    return pl.pallas_call(
        _tc_pre1_body,
        grid=grid,
        in_specs=[
            pl.BlockSpec((BN, IN_DIM), lambda i: (i, 0)),
            pl.BlockSpec((IN_DIM, F1), lambda i: (0, 0)),
            pl.BlockSpec((HEADS, F1), lambda i: (0, 0)),
            pl.BlockSpec((HEADS, F1), lambda i: (0, 0)),
            pl.BlockSpec((F1, HEADS), lambda i: (0, 0)),
        ],
        out_specs=[
            pl.BlockSpec((F1, BN), lambda i: (0, i)),
            pl.BlockSpec((HEADS, BN), lambda i: (0, i)),
            pl.BlockSpec((HEADS, BN), lambda i: (0, i)),
            pl.BlockSpec((F1, BN), lambda i: (0, i)),
            pl.BlockSpec((HEADS, BN), lambda i: (0, i)),
        ],
        out_shape=[
            jax.ShapeDtypeStruct((F1, NP), jnp.float32),
            jax.ShapeDtypeStruct((HEADS, NP), jnp.float32),
            jax.ShapeDtypeStruct((HEADS, NP), jnp.float32),
            jax.ShapeDtypeStruct((F1, NP), jnp.float32),
            jax.ShapeDtypeStruct((HEADS, NP), jnp.float32),
        ],
    )(x_pad, W1, A_src, A_dst, Sel)


_SC_PARAMS = pltpu.CompilerParams(
    needs_layout_passes=False, use_tc_tiling_on_sc=False
)
_MESH = dict(core_axis_name="c", subcore_axis_name="s")


def _sc_edge1(src, dst, a_srcT, a_dstT, h1T, num_init, den_init):
    mesh = plsc.VectorSubcoreMesh(**_MESH)

    @functools.partial(
        pl.kernel,
        out_type=[
            jax.ShapeDtypeStruct((F1, NP), jnp.float32),
            jax.ShapeDtypeStruct((HEADS, NP), jnp.float32),
        ],
        mesh=mesh,
        scratch_types=[
            pltpu.VMEM((4, NP), jnp.float32),
            pltpu.VMEM((4, NP), jnp.float32),
            pltpu.VMEM((1, NP), jnp.float32),
            pltpu.VMEM((1, NP), jnp.float32),
            pltpu.VMEM((1, NP), jnp.float32),
            pltpu.VMEM((CH,), jnp.int32),
            pltpu.VMEM((CH,), jnp.int32),
            pltpu.VMEM((CH,), jnp.int32),
            pltpu.VMEM((CH,), jnp.int32),
            pltpu.SemaphoreType.DMA,
            pltpu.SemaphoreType.DMA,
        ],
        compiler_params=_SC_PARAMS,
    )
    def k(src_h, dst_h, as_h, ad_h, h1_h, n0_h, d0_h, num_o, den_o,
          hc, acc, asr, adr, dacc, sb0, db0, sb1, db1, sem0, sem1):
        t = lax.axis_index("s") * 2 + lax.axis_index("c")
        head = t // 8
        pltpu.sync_copy(h1_h.at[pl.ds(4 * t, 4)], hc)
        pltpu.sync_copy(n0_h.at[pl.ds(4 * t, 4)], acc)
        pltpu.sync_copy(as_h.at[pl.ds(head, 1)], asr)
        pltpu.sync_copy(ad_h.at[pl.ds(head, 1)], adr)
        pltpu.sync_copy(d0_h.at[pl.ds(head, 1)], dacc)

        f0 = jnp.full((16,), 0, jnp.int32)
        f1 = jnp.full((16,), 1, jnp.int32)
        f2 = jnp.full((16,), 2, jnp.int32)
        f3 = jnp.full((16,), 3, jnp.int32)

        def start(c, sb, db, sem):
            pltpu.async_copy(src_h.at[pl.ds(c * CH, CH)], sb, sem)
            pltpu.async_copy(dst_h.at[pl.ds(c * CH, CH)], db, sem)

        def drain(sb, db, sem):
            pltpu.make_async_copy(src_h.at[pl.ds(0, CH)], sb, sem).wait()
            pltpu.make_async_copy(dst_h.at[pl.ds(0, CH)], db, sem).wait()

        def inner(sb, db):
            U = 4

            def blk(i, _):
                for u in range(U):
                    o = (i * U + u) * 16
                    sv = sb[pl.ds(o, 16)]
                    dv = db[pl.ds(o, 16)]
                    e = (plsc.load_gather(asr, [f0, sv])
                         + plsc.load_gather(adr, [f0, dv]))
                    p = jnp.exp(jnp.maximum(e, 0.2 * e))
                    plsc.addupdate_scatter(dacc, [f0, dv], p)
                    for fv in (f0, f1, f2, f3):
                        hv = plsc.load_gather(hc, [fv, sv])
                        plsc.addupdate_scatter(acc, [fv, dv], hv * p)
                return 0

            lax.fori_loop(0, CH // (16 * U), blk, 0)

        start(0, sb0, db0, sem0)

        def chunk_body(g2, _):
            c = 2 * g2
            start(c + 1, sb1, db1, sem1)
            drain(sb0, db0, sem0)
            inner(sb0, db0)

            @pl.when(c + 2 < E // CH)
            def _():
                start(c + 2, sb0, db0, sem0)

            drain(sb1, db1, sem1)
            inner(sb1, db1)
            return 0

        lax.fori_loop(0, E // (2 * CH), chunk_body, 0)

        pltpu.sync_copy(acc, num_o.at[pl.ds(4 * t, 4)])

        @pl.when(t % 8 == 0)
        def _():
            pltpu.sync_copy(dacc, den_o.at[pl.ds(head, 1)])

    return k(src, dst, a_srcT, a_dstT, h1T, num_init, den_init)


def _tc_mid_body(num_ref, den_ref, b1_ref, w2_ref, as2_ref, ad2_ref, sel_ref,
                 h2t_ref, as_ref, ad_ref, num0_ref, den0_ref):
    den_full = lax.dot_general(sel_ref[...], den_ref[...] + 1e-16,
                               (((1,), (0,)), ((), ())),
                               preferred_element_type=jnp.float32)
    out1 = num_ref[...] / den_full + b1_ref[...]
    out1 = _elu(out1)
    h2t = lax.dot_general(w2_ref[...], out1, (((0,), (0,)), ((), ())),
                          preferred_element_type=jnp.float32)
    a_s = lax.dot_general(as2_ref[...], h2t, (((1,), (0,)), ((), ())),
                          preferred_element_type=jnp.float32)
    a_d = lax.dot_general(ad2_ref[...], h2t, (((1,), (0,)), ((), ())),
                          preferred_element_type=jnp.float32)
    p_self = jnp.exp(_lrelu(a_s + a_d))
    h2t_ref[...] = h2t
    as_ref[...] = a_s
    ad_ref[...] = a_d
    num0_ref[...] = h2t * p_self
    den0_ref[...] = p_self


def _tc_mid(num1, den1, b1c, W2, a2s, a2d, Sel):
    grid = (NP // BN,)
    return pl.pallas_call(
        _tc_mid_body,
        grid=grid,
        in_specs=[
            pl.BlockSpec((F1, BN), lambda i: (0, i)),
            pl.BlockSpec((HEADS, BN), lambda i: (0, i)),
            pl.BlockSpec((F1, 1), lambda i: (0, 0)),
            pl.BlockSpec((F1, F2), lambda i: (0, 0)),
            pl.BlockSpec((1, F2), lambda i: (0, 0)),
            pl.BlockSpec((1, F2), lambda i: (0, 0)),
            pl.BlockSpec((F1, HEADS), lambda i: (0, 0)),
        ],
        out_specs=[
            pl.BlockSpec((F2, BN), lambda i: (0, i)),
            pl.BlockSpec((1, BN), lambda i: (0, i)),
            pl.BlockSpec((1, BN), lambda i: (0, i)),
            pl.BlockSpec((F2, BN), lambda i: (0, i)),
            pl.BlockSpec((1, BN), lambda i: (0, i)),
        ],
        out_shape=[
            jax.ShapeDtypeStruct((F2, NP), jnp.float32),
            jax.ShapeDtypeStruct((1, NP), jnp.float32),
            jax.ShapeDtypeStruct((1, NP), jnp.float32),
            jax.ShapeDtypeStruct((F2, NP), jnp.float32),
            jax.ShapeDtypeStruct((1, NP), jnp.float32),
        ],
    )(num1, den1, b1c, W2, a2s, a2d, Sel)


def _sc_edge2(src, dst, a_src2, a_dst2, h2T, num_init, den_init):
    mesh = plsc.VectorSubcoreMesh(**_MESH)

    @functools.partial(
        pl.kernel,
        out_type=[
            jax.ShapeDtypeStruct((F2, NP), jnp.float32),
            jax.ShapeDtypeStruct((1, NP), jnp.float32),
        ],
        mesh=mesh,
        scratch_types=[
            pltpu.VMEM((1, NP), jnp.float32),
            pltpu.VMEM((1, NP), jnp.float32),
            pltpu.VMEM((1, NP), jnp.float32),
            pltpu.VMEM((1, NP), jnp.float32),
            pltpu.VMEM((1, NP), jnp.float32),
            pltpu.VMEM((CH,), jnp.int32),
            pltpu.VMEM((CH,), jnp.int32),
            pltpu.VMEM((CH,), jnp.int32),
            pltpu.VMEM((CH,), jnp.int32),
            pltpu.SemaphoreType.DMA,
            pltpu.SemaphoreType.DMA,
        ],
        compiler_params=_SC_PARAMS,
    )
    def k(src_h, dst_h, as_h, ad_h, h2_h, n0_h, d0_h, num_o, den_o,
          hc, acc, asr, adr, dacc, sb0, db0, sb1, db1, sem0, sem1):
        t = lax.axis_index("s") * 2 + lax.axis_index("c")
        pltpu.sync_copy(h2_h.at[pl.ds(t, 1)], hc)
        pltpu.sync_copy(n0_h.at[pl.ds(t, 1)], acc)
        pltpu.sync_copy(as_h, asr)
        pltpu.sync_copy(ad_h, adr)
        pltpu.sync_copy(d0_h, dacc)

        f0 = jnp.full((16,), 0, jnp.int32)

        def start(c, sb, db, sem):
            pltpu.async_copy(src_h.at[pl.ds(c * CH, CH)], sb, sem)
            pltpu.async_copy(dst_h.at[pl.ds(c * CH, CH)], db, sem)

        def drain(sb, db, sem):
            pltpu.make_async_copy(src_h.at[pl.ds(0, CH)], sb, sem).wait()
            pltpu.make_async_copy(dst_h.at[pl.ds(0, CH)], db, sem).wait()

        def inner(sb, db):
            U = 4

            def blk(i, _):
                for u in range(U):
                    o = (i * U + u) * 16
                    sv = sb[pl.ds(o, 16)]
                    dv = db[pl.ds(o, 16)]
                    e = (plsc.load_gather(asr, [f0, sv])
                         + plsc.load_gather(adr, [f0, dv]))
                    p = jnp.exp(jnp.maximum(e, 0.2 * e))
                    plsc.addupdate_scatter(dacc, [f0, dv], p)
                    hv = plsc.load_gather(hc, [f0, sv])
                    plsc.addupdate_scatter(acc, [f0, dv], hv * p)
                return 0

            lax.fori_loop(0, CH // (16 * U), blk, 0)

        start(0, sb0, db0, sem0)

        def chunk_body(g2, _):
            c = 2 * g2
            start(c + 1, sb1, db1, sem1)
            drain(sb0, db0, sem0)
            inner(sb0, db0)

            @pl.when(c + 2 < E // CH)
            def _():
                start(c + 2, sb0, db0, sem0)

            drain(sb1, db1, sem1)
            inner(sb1, db1)
            return 0

        lax.fori_loop(0, E // (2 * CH), chunk_body, 0)

        pltpu.sync_copy(acc, num_o.at[pl.ds(t, 1)])

        @pl.when(t == 0)
        def _():
            pltpu.sync_copy(dacc, den_o)

    return k(src, dst, a_src2, a_dst2, h2T, num_init, den_init)


def _tc_post_body(num_ref, den_ref, b2_ref, lw_ref, lb_ref, out_ref):
    out2 = num_ref[...] / (den_ref[...] + 1e-16) + b2_ref[...]
    out2 = _elu(out2)
    y = lax.dot_general(lw_ref[...], out2, (((0,), (0,)), ((), ())),
                        preferred_element_type=jnp.float32)
    out_ref[...] = y + lb_ref[...]


def _tc_post(num2, den2, b2c, lin_W, lin_bc):
    grid = (NP // BN,)
    return pl.pallas_call(
        _tc_post_body,
        grid=grid,
        in_specs=[
            pl.BlockSpec((F2, BN), lambda i: (0, i)),
            pl.BlockSpec((1, BN), lambda i: (0, i)),
            pl.BlockSpec((F2, 1), lambda i: (0, 0)),
            pl.BlockSpec((F2, OUT), lambda i: (0, 0)),
            pl.BlockSpec((OUT, 1), lambda i: (0, 0)),
        ],
        out_specs=pl.BlockSpec((OUT, BN), lambda i: (0, i)),
        out_shape=jax.ShapeDtypeStruct((OUT, NP), jnp.float32),
    )(num2, den2, b2c, lin_W, lin_bc)


def kernel(x, edge_index, W1, att_src1, att_dst1, b1, W2, att_src2, att_dst2,
           b2, lin_W, lin_b):
    x_pad = jnp.pad(x, ((0, NP - N), (0, 0)))
    src = edge_index[0]
    dst = edge_index[1]

    head_of = np.arange(F1) // HID
    blockmask = jnp.asarray(
        (np.arange(HEADS)[:, None] == head_of[None, :]).astype(np.float32))
    A_src = blockmask * att_src1.reshape(1, F1)
    A_dst = blockmask * att_dst1.reshape(1, F1)
    Sel = blockmask.T

    h1T, a_sT, a_dT, num0, den0 = _tc_pre1(x_pad, W1, A_src, A_dst, Sel)
    num1, den1 = _sc_edge1(src, dst, a_sT, a_dT, h1T, num0, den0)

    h2T, a2sT, a2dT, num0b, den0b = _tc_mid(
        num1, den1, b1.reshape(F1, 1), W2,
        att_src2.reshape(1, F2), att_dst2.reshape(1, F2), Sel)
    num2, den2 = _sc_edge2(src, dst, a2sT, a2dT, h2T, num0b, den0b)

    yT = _tc_post(num2, den2, b2.reshape(F2, 1), lin_W,
                  lin_b.reshape(OUT, 1))
    return yT[:, :N].T


# speedup vs baseline: 29.1541x; 1.1658x over previous
import functools

import jax
import jax.numpy as jnp
import numpy as np
from jax import lax
from jax.experimental import pallas as pl
from jax.experimental.pallas import tpu as pltpu, tpu_sc as plsc

N = 10000
NP = 10240
E = 320000
IN_DIM = 128
HID = 32
HEADS = 4
F1 = HEADS * HID
F2 = HID
OUT = 2
BN = 2048
CH = 3200


def _lrelu(x):
    return jnp.maximum(x, 0.2 * x)


def _elu(x):
    return jnp.where(x > 0, x, jnp.exp(jnp.minimum(x, 0.0)) - 1.0)


def _tc_pre1_body(x_ref, w1_ref, asrc_ref, adst_ref, sel_ref,
                  h1t_ref, as_ref, ad_ref, num0_ref, den0_ref):
    xb = x_ref[...]
    h1t = lax.dot_general(w1_ref[...], xb, (((0,), (1,)), ((), ())),
                          preferred_element_type=jnp.float32)
    a_s = lax.dot_general(asrc_ref[...], h1t, (((1,), (0,)), ((), ())),
                          preferred_element_type=jnp.float32)
    a_d = lax.dot_general(adst_ref[...], h1t, (((1,), (0,)), ((), ())),
                          preferred_element_type=jnp.float32)
    p_self = jnp.exp(_lrelu(a_s + a_d))
    p_full = lax.dot_general(sel_ref[...], p_self, (((1,), (0,)), ((), ())),
                             preferred_element_type=jnp.float32)
    h1t_ref[...] = h1t
    as_ref[...] = a_s
    ad_ref[...] = a_d
    num0_ref[...] = h1t * p_full
    den0_ref[...] = p_self


def _tc_pre1(x_pad, W1, A_src, A_dst, Sel):
    grid = (NP // BN,)
    return pl.pallas_call(
        _tc_pre1_body,
        grid=grid,
        in_specs=[
            pl.BlockSpec((BN, IN_DIM), lambda i: (i, 0)),
            pl.BlockSpec((IN_DIM, F1), lambda i: (0, 0)),
            pl.BlockSpec((HEADS, F1), lambda i: (0, 0)),
            pl.BlockSpec((HEADS, F1), lambda i: (0, 0)),
            pl.BlockSpec((F1, HEADS), lambda i: (0, 0)),
        ],
        out_specs=[
            pl.BlockSpec((F1, BN), lambda i: (0, i)),
            pl.BlockSpec((HEADS, BN), lambda i: (0, i)),
            pl.BlockSpec((HEADS, BN), lambda i: (0, i)),
            pl.BlockSpec((F1, BN), lambda i: (0, i)),
            pl.BlockSpec((HEADS, BN), lambda i: (0, i)),
        ],
        out_shape=[
            jax.ShapeDtypeStruct((F1, NP), jnp.float32),
            jax.ShapeDtypeStruct((HEADS, NP), jnp.float32),
            jax.ShapeDtypeStruct((HEADS, NP), jnp.float32),
            jax.ShapeDtypeStruct((F1, NP), jnp.float32),
            jax.ShapeDtypeStruct((HEADS, NP), jnp.float32),
        ],
    )(x_pad, W1, A_src, A_dst, Sel)


_SC_PARAMS = pltpu.CompilerParams(
    needs_layout_passes=False, use_tc_tiling_on_sc=False
)
_MESH = dict(core_axis_name="c", subcore_axis_name="s")


def _sc_edge1(src, dst, a_srcT, a_dstT, h1T, num_init, den_init):
    mesh = plsc.VectorSubcoreMesh(**_MESH)

    @functools.partial(
        pl.kernel,
        out_type=[
            jax.ShapeDtypeStruct((F1, NP), jnp.float32),
            jax.ShapeDtypeStruct((HEADS, NP), jnp.float32),
        ],
        mesh=mesh,
        scratch_types=[
            pltpu.VMEM((4, NP), jnp.float32),
            pltpu.VMEM((4, NP), jnp.float32),
            pltpu.VMEM((1, NP), jnp.float32),
            pltpu.VMEM((1, NP), jnp.float32),
            pltpu.VMEM((1, NP), jnp.float32),
            pltpu.VMEM((CH,), jnp.int32),
            pltpu.VMEM((CH,), jnp.int32),
            pltpu.VMEM((CH,), jnp.int32),
            pltpu.VMEM((CH,), jnp.int32),
            pltpu.SemaphoreType.DMA,
            pltpu.SemaphoreType.DMA,
        ],
        compiler_params=_SC_PARAMS,
    )
    def k(src_h, dst_h, as_h, ad_h, h1_h, n0_h, d0_h, num_o, den_o,
          hc, acc, asr, adr, dacc, sb0, db0, sb1, db1, sem0, sem1):
        t = lax.axis_index("s") * 2 + lax.axis_index("c")
        head = t // 8
        pltpu.sync_copy(h1_h.at[pl.ds(4 * t, 4)], hc)
        pltpu.sync_copy(n0_h.at[pl.ds(4 * t, 4)], acc)
        pltpu.sync_copy(as_h.at[pl.ds(head, 1)], asr)
        pltpu.sync_copy(ad_h.at[pl.ds(head, 1)], adr)
        pltpu.sync_copy(d0_h.at[pl.ds(head, 1)], dacc)

        f0 = jnp.full((16,), 0, jnp.int32)
        f1 = jnp.full((16,), 1, jnp.int32)
        f2 = jnp.full((16,), 2, jnp.int32)
        f3 = jnp.full((16,), 3, jnp.int32)

        def start(c, sb, db, sem):
            pltpu.async_copy(src_h.at[pl.ds(c * CH, CH)], sb, sem)
            pltpu.async_copy(dst_h.at[pl.ds(c * CH, CH)], db, sem)

        def drain(sb, db, sem):
            pltpu.make_async_copy(src_h.at[pl.ds(0, CH)], sb, sem).wait()
            pltpu.make_async_copy(dst_h.at[pl.ds(0, CH)], db, sem).wait()

        def inner(sb, db):
            U = 8

            def blk(i, _):
                for u in range(U):
                    o = (i * U + u) * 16
                    sv = sb[pl.ds(o, 16)]
                    dv = db[pl.ds(o, 16)]
                    e = (plsc.load_gather(asr, [f0, sv])
                         + plsc.load_gather(adr, [f0, dv]))
                    p = jnp.exp(jnp.maximum(e, 0.2 * e))
                    plsc.addupdate_scatter(dacc, [f0, dv], p)
                    for fv in (f0, f1, f2, f3):
                        hv = plsc.load_gather(hc, [fv, sv])
                        plsc.addupdate_scatter(acc, [fv, dv], hv * p)
                return 0

            lax.fori_loop(0, CH // (16 * U), blk, 0)

        start(0, sb0, db0, sem0)

        def chunk_body(g2, _):
            c = 2 * g2
            start(c + 1, sb1, db1, sem1)
            drain(sb0, db0, sem0)
            inner(sb0, db0)

            @pl.when(c + 2 < E // CH)
            def _():
                start(c + 2, sb0, db0, sem0)

            drain(sb1, db1, sem1)
            inner(sb1, db1)
            return 0

        lax.fori_loop(0, E // (2 * CH), chunk_body, 0)

        pltpu.sync_copy(acc, num_o.at[pl.ds(4 * t, 4)])

        @pl.when(t % 8 == 0)
        def _():
            pltpu.sync_copy(dacc, den_o.at[pl.ds(head, 1)])

    return k(src, dst, a_srcT, a_dstT, h1T, num_init, den_init)


def _tc_mid_body(num_ref, den_ref, b1_ref, w2_ref, as2_ref, ad2_ref, sel_ref,
                 h2t_ref, as_ref, ad_ref, num0_ref, den0_ref):
    den_full = lax.dot_general(sel_ref[...], den_ref[...] + 1e-16,
                               (((1,), (0,)), ((), ())),
                               preferred_element_type=jnp.float32)
    out1 = num_ref[...] / den_full + b1_ref[...]
    out1 = _elu(out1)
    h2t = lax.dot_general(w2_ref[...], out1, (((0,), (0,)), ((), ())),
                          preferred_element_type=jnp.float32)
    a_s = lax.dot_general(as2_ref[...], h2t, (((1,), (0,)), ((), ())),
                          preferred_element_type=jnp.float32)
    a_d = lax.dot_general(ad2_ref[...], h2t, (((1,), (0,)), ((), ())),
                          preferred_element_type=jnp.float32)
    p_self = jnp.exp(_lrelu(a_s + a_d))
    h2t_ref[...] = h2t
    as_ref[...] = a_s
    ad_ref[...] = a_d
    num0_ref[...] = h2t * p_self
    den0_ref[...] = p_self


def _tc_mid(num1, den1, b1c, W2, a2s, a2d, Sel):
    grid = (NP // BN,)
    return pl.pallas_call(
        _tc_mid_body,
        grid=grid,
        in_specs=[
            pl.BlockSpec((F1, BN), lambda i: (0, i)),
            pl.BlockSpec((HEADS, BN), lambda i: (0, i)),
            pl.BlockSpec((F1, 1), lambda i: (0, 0)),
            pl.BlockSpec((F1, F2), lambda i: (0, 0)),
            pl.BlockSpec((1, F2), lambda i: (0, 0)),
            pl.BlockSpec((1, F2), lambda i: (0, 0)),
            pl.BlockSpec((F1, HEADS), lambda i: (0, 0)),
        ],
        out_specs=[
            pl.BlockSpec((F2, BN), lambda i: (0, i)),
            pl.BlockSpec((1, BN), lambda i: (0, i)),
            pl.BlockSpec((1, BN), lambda i: (0, i)),
            pl.BlockSpec((F2, BN), lambda i: (0, i)),
            pl.BlockSpec((1, BN), lambda i: (0, i)),
        ],
        out_shape=[
            jax.ShapeDtypeStruct((F2, NP), jnp.float32),
            jax.ShapeDtypeStruct((1, NP), jnp.float32),
            jax.ShapeDtypeStruct((1, NP), jnp.float32),
            jax.ShapeDtypeStruct((F2, NP), jnp.float32),
            jax.ShapeDtypeStruct((1, NP), jnp.float32),
        ],
    )(num1, den1, b1c, W2, a2s, a2d, Sel)


def _sc_edge2(src, dst, a_src2, a_dst2, h2T, num_init, den_init):
    mesh = plsc.VectorSubcoreMesh(**_MESH)
    EH = E // 2

    @functools.partial(
        pl.kernel,
        out_type=[
            jax.ShapeDtypeStruct((2, F2, NP), jnp.float32),
            jax.ShapeDtypeStruct((2, NP), jnp.float32),
        ],
        mesh=mesh,
        scratch_types=[
            pltpu.VMEM((2, NP), jnp.float32),
            pltpu.VMEM((2, NP), jnp.float32),
            pltpu.VMEM((1, NP), jnp.float32),
            pltpu.VMEM((1, NP), jnp.float32),
            pltpu.VMEM((1, NP), jnp.float32),
            pltpu.VMEM((CH,), jnp.int32),
            pltpu.VMEM((CH,), jnp.int32),
            pltpu.VMEM((CH,), jnp.int32),
            pltpu.VMEM((CH,), jnp.int32),
            pltpu.SemaphoreType.DMA,
            pltpu.SemaphoreType.DMA,
        ],
        compiler_params=_SC_PARAMS,
    )
    def k(src_h, dst_h, as_h, ad_h, h2_h, n0_h, d0_h, num_o, den_o,
          hc, acc, asr, adr, dacc, sb0, db0, sb1, db1, sem0, sem1):
        t = lax.axis_index("s") * 2 + lax.axis_index("c")
        half = t // 16
        fp = t % 16
        ebase = half * EH
        pltpu.sync_copy(h2_h.at[pl.ds(2 * fp, 2)], hc)
        pltpu.sync_copy(as_h, asr)
        pltpu.sync_copy(ad_h, adr)

        @pl.when(half == 0)
        def _():
            pltpu.sync_copy(n0_h.at[pl.ds(2 * fp, 2)], acc)
            pltpu.sync_copy(d0_h, dacc)

        @pl.when(half == 1)
        def _():
            z = jnp.zeros((16,), jnp.float32)

            def zb(i, _):
                acc[0, pl.ds(i * 16, 16)] = z
                acc[1, pl.ds(i * 16, 16)] = z
                dacc[0, pl.ds(i * 16, 16)] = z
                return 0

            lax.fori_loop(0, NP // 16, zb, 0)

        f0 = jnp.full((16,), 0, jnp.int32)
        f1 = jnp.full((16,), 1, jnp.int32)

        def start(c, sb, db, sem):
            pltpu.async_copy(src_h.at[pl.ds(ebase + c * CH, CH)], sb, sem)
            pltpu.async_copy(dst_h.at[pl.ds(ebase + c * CH, CH)], db, sem)

        def drain(sb, db, sem):
            pltpu.make_async_copy(src_h.at[pl.ds(0, CH)], sb, sem).wait()
            pltpu.make_async_copy(dst_h.at[pl.ds(0, CH)], db, sem).wait()

        def inner(sb, db):
            U = 8

            def blk(i, _):
                for u in range(U):
                    o = (i * U + u) * 16
                    sv = sb[pl.ds(o, 16)]
                    dv = db[pl.ds(o, 16)]
                    e = (plsc.load_gather(asr, [f0, sv])
                         + plsc.load_gather(adr, [f0, dv]))
                    p = jnp.exp(jnp.maximum(e, 0.2 * e))
                    plsc.addupdate_scatter(dacc, [f0, dv], p)
                    for fv in (f0, f1):
                        hv = plsc.load_gather(hc, [fv, sv])
                        plsc.addupdate_scatter(acc, [fv, dv], hv * p)
                return 0

            lax.fori_loop(0, CH // (16 * U), blk, 0)

        start(0, sb0, db0, sem0)

        def chunk_body(g2, _):
            c = 2 * g2
            start(c + 1, sb1, db1, sem1)
            drain(sb0, db0, sem0)
            inner(sb0, db0)

            @pl.when(c + 2 < EH // CH)
            def _():
                start(c + 2, sb0, db0, sem0)

            drain(sb1, db1, sem1)
            inner(sb1, db1)
            return 0

        lax.fori_loop(0, EH // (2 * CH), chunk_body, 0)

        pltpu.sync_copy(acc, num_o.at[half, pl.ds(2 * fp, 2)])

        @pl.when(fp == 0)
        def _():
            pltpu.sync_copy(dacc, den_o.at[pl.ds(half, 1)])

    return k(src, dst, a_src2, a_dst2, h2T, num_init, den_init)


def _tc_post_body(num_ref, den_ref, b2_ref, lw_ref, lb_ref, out_ref):
    num = num_ref[0] + num_ref[1]
    den = den_ref[pl.ds(0, 1)] + den_ref[pl.ds(1, 1)]
    out2 = num / (den + 1e-16) + b2_ref[...]
    out2 = _elu(out2)
    y = lax.dot_general(lw_ref[...], out2, (((0,), (0,)), ((), ())),
                        preferred_element_type=jnp.float32)
    out_ref[...] = y + lb_ref[...]


def _tc_post(num2, den2, b2c, lin_W, lin_bc):
    grid = (NP // BN,)
    return pl.pallas_call(
        _tc_post_body,
        grid=grid,
        in_specs=[
            pl.BlockSpec((2, F2, BN), lambda i: (0, 0, i)),
            pl.BlockSpec((2, BN), lambda i: (0, i)),
            pl.BlockSpec((F2, 1), lambda i: (0, 0)),
            pl.BlockSpec((F2, OUT), lambda i: (0, 0)),
            pl.BlockSpec((OUT, 1), lambda i: (0, 0)),
        ],
        out_specs=pl.BlockSpec((OUT, BN), lambda i: (0, i)),
        out_shape=jax.ShapeDtypeStruct((OUT, NP), jnp.float32),
    )(num2, den2, b2c, lin_W, lin_bc)


def kernel(x, edge_index, W1, att_src1, att_dst1, b1, W2, att_src2, att_dst2,
           b2, lin_W, lin_b):
    x_pad = jnp.pad(x, ((0, NP - N), (0, 0)))
    src = edge_index[0]
    dst = edge_index[1]

    head_of = np.arange(F1) // HID
    blockmask = jnp.asarray(
        (np.arange(HEADS)[:, None] == head_of[None, :]).astype(np.float32))
    A_src = blockmask * att_src1.reshape(1, F1)
    A_dst = blockmask * att_dst1.reshape(1, F1)
    Sel = blockmask.T

    h1T, a_sT, a_dT, num0, den0 = _tc_pre1(x_pad, W1, A_src, A_dst, Sel)
    num1, den1 = _sc_edge1(src, dst, a_sT, a_dT, h1T, num0, den0)

    h2T, a2sT, a2dT, num0b, den0b = _tc_mid(
        num1, den1, b1.reshape(F1, 1), W2,
        att_src2.reshape(1, F2), att_dst2.reshape(1, F2), Sel)
    num2, den2 = _sc_edge2(src, dst, a2sT, a2dT, h2T, num0b, den0b)

    yT = _tc_post(num2, den2, b2.reshape(F2, 1), lin_W,
                  lin_b.reshape(OUT, 1))
    return yT[:, :N].T


# speedup vs baseline: 65.1414x; 2.2344x over previous
import functools

import jax
import jax.numpy as jnp
import numpy as np
from jax import lax
from jax.experimental import pallas as pl
from jax.experimental.pallas import tpu as pltpu, tpu_sc as plsc

N = 10000
NP = 10240
E = 320000
IN_DIM = 128
HID = 32
HEADS = 4
F1 = HEADS * HID
F2 = HID
OUT = 2
BN = 2048
CH = 3200


def _lrelu(x):
    return jnp.maximum(x, 0.2 * x)


def _elu(x):
    return jnp.where(x > 0, x, jnp.exp(jnp.minimum(x, 0.0)) - 1.0)


def _tc_pre1_body(x_ref, w1_ref, asrc_ref, adst_ref, sel_ref,
                  h1t_ref, as_ref, ad_ref, num0_ref, den0_ref):
    xb = x_ref[...]
    h1t = lax.dot_general(w1_ref[...], xb, (((0,), (1,)), ((), ())),
                          preferred_element_type=jnp.float32)
    a_s = lax.dot_general(asrc_ref[...], h1t, (((1,), (0,)), ((), ())),
                          preferred_element_type=jnp.float32)
    a_d = lax.dot_general(adst_ref[...], h1t, (((1,), (0,)), ((), ())),
                          preferred_element_type=jnp.float32)
    p_self = jnp.exp(_lrelu(a_s + a_d))
    p_full = lax.dot_general(sel_ref[...], p_self, (((1,), (0,)), ((), ())),
                             preferred_element_type=jnp.float32)
    h1t_ref[...] = h1t
    as_ref[...] = a_s
    ad_ref[...] = a_d
    num0_ref[...] = h1t * p_full
    den0_ref[...] = p_self


def _tc_pre1(x_pad, W1, A_src, A_dst, Sel):
    grid = (NP // BN,)
    return pl.pallas_call(
        _tc_pre1_body,
        grid=grid,
        in_specs=[
            pl.BlockSpec((BN, IN_DIM), lambda i: (i, 0)),
            pl.BlockSpec((IN_DIM, F1), lambda i: (0, 0)),
            pl.BlockSpec((HEADS, F1), lambda i: (0, 0)),
            pl.BlockSpec((HEADS, F1), lambda i: (0, 0)),
            pl.BlockSpec((F1, HEADS), lambda i: (0, 0)),
        ],
        out_specs=[
            pl.BlockSpec((F1, BN), lambda i: (0, i)),
            pl.BlockSpec((HEADS, BN), lambda i: (0, i)),
            pl.BlockSpec((HEADS, BN), lambda i: (0, i)),
            pl.BlockSpec((F1, BN), lambda i: (0, i)),
            pl.BlockSpec((HEADS, BN), lambda i: (0, i)),
        ],
        out_shape=[
            jax.ShapeDtypeStruct((F1, NP), jnp.float32),
            jax.ShapeDtypeStruct((HEADS, NP), jnp.float32),
            jax.ShapeDtypeStruct((HEADS, NP), jnp.float32),
            jax.ShapeDtypeStruct((F1, NP), jnp.float32),
            jax.ShapeDtypeStruct((HEADS, NP), jnp.float32),
        ],
    )(x_pad, W1, A_src, A_dst, Sel)


_SC_PARAMS = pltpu.CompilerParams(
    needs_layout_passes=False, use_tc_tiling_on_sc=False
)
_MESH = dict(core_axis_name="c", subcore_axis_name="s")


def _sc_edge1(src, dst, a_srcT, a_dstT, h1T, num_init, den_init):
    mesh = plsc.VectorSubcoreMesh(**_MESH)

    @functools.partial(
        pl.kernel,
        out_type=[
            jax.ShapeDtypeStruct((F1, NP), jnp.float32),
            jax.ShapeDtypeStruct((HEADS, NP), jnp.float32),
        ],
        mesh=mesh,
        scratch_types=[
            pltpu.VMEM((4, NP), jnp.float32),
            pltpu.VMEM((4, NP), jnp.float32),
            pltpu.VMEM((1, NP), jnp.float32),
            pltpu.VMEM((1, NP), jnp.float32),
            pltpu.VMEM((1, NP), jnp.float32),
            pltpu.VMEM((CH,), jnp.int32),
            pltpu.VMEM((CH,), jnp.int32),
            pltpu.VMEM((CH,), jnp.int32),
            pltpu.VMEM((CH,), jnp.int32),
            pltpu.SemaphoreType.DMA,
            pltpu.SemaphoreType.DMA,
        ],
        compiler_params=_SC_PARAMS,
    )
    def k(src_h, dst_h, as_h, ad_h, h1_h, n0_h, d0_h, num_o, den_o,
          hc, acc, asr, adr, dacc, sb0, db0, sb1, db1, sem0, sem1):
        t = lax.axis_index("s") * 2 + lax.axis_index("c")
        head = t // 8
        pltpu.sync_copy(h1_h.at[pl.ds(4 * t, 4)], hc)
        pltpu.sync_copy(n0_h.at[pl.ds(4 * t, 4)], acc)
        pltpu.sync_copy(as_h.at[pl.ds(head, 1)], asr)
        pltpu.sync_copy(ad_h.at[pl.ds(head, 1)], adr)
        pltpu.sync_copy(d0_h.at[pl.ds(head, 1)], dacc)

        f0 = jnp.full((16,), 0, jnp.int32)
        f1 = jnp.full((16,), 1, jnp.int32)
        f2 = jnp.full((16,), 2, jnp.int32)
        f3 = jnp.full((16,), 3, jnp.int32)

        def start(c, sb, db, sem):
            pltpu.async_copy(src_h.at[pl.ds(c * CH, CH)], sb, sem)
            pltpu.async_copy(dst_h.at[pl.ds(c * CH, CH)], db, sem)

        def drain(sb, db, sem):
            pltpu.make_async_copy(src_h.at[pl.ds(0, CH)], sb, sem).wait()
            pltpu.make_async_copy(dst_h.at[pl.ds(0, CH)], db, sem).wait()

        def inner(sb, db):
            @plsc.parallel_loop(0, CH // 16, unroll=8)
            def blk(i):
                o = i * 16
                sv = sb[pl.ds(o, 16)]
                dv = db[pl.ds(o, 16)]
                e = (plsc.load_gather(asr, [f0, sv])
                     + plsc.load_gather(adr, [f0, dv]))
                p = jnp.exp(jnp.maximum(e, 0.2 * e))
                plsc.addupdate_scatter(dacc, [f0, dv], p)
                for fv in (f0, f1, f2, f3):
                    hv = plsc.load_gather(hc, [fv, sv])
                    plsc.addupdate_scatter(acc, [fv, dv], hv * p)

        start(0, sb0, db0, sem0)

        def chunk_body(g2, _):
            c = 2 * g2
            start(c + 1, sb1, db1, sem1)
            drain(sb0, db0, sem0)
            inner(sb0, db0)

            @pl.when(c + 2 < E // CH)
            def _():
                start(c + 2, sb0, db0, sem0)

            drain(sb1, db1, sem1)
            inner(sb1, db1)
            return 0

        lax.fori_loop(0, E // (2 * CH), chunk_body, 0)

        pltpu.sync_copy(acc, num_o.at[pl.ds(4 * t, 4)])

        @pl.when(t % 8 == 0)
        def _():
            pltpu.sync_copy(dacc, den_o.at[pl.ds(head, 1)])

    return k(src, dst, a_srcT, a_dstT, h1T, num_init, den_init)


def _tc_mid_body(num_ref, den_ref, b1_ref, w2_ref, as2_ref, ad2_ref, sel_ref,
                 h2t_ref, as_ref, ad_ref, num0_ref, den0_ref):
    den_full = lax.dot_general(sel_ref[...], den_ref[...] + 1e-16,
                               (((1,), (0,)), ((), ())),
                               preferred_element_type=jnp.float32)
    out1 = num_ref[...] / den_full + b1_ref[...]
    out1 = _elu(out1)
    h2t = lax.dot_general(w2_ref[...], out1, (((0,), (0,)), ((), ())),
                          preferred_element_type=jnp.float32)
    a_s = lax.dot_general(as2_ref[...], h2t, (((1,), (0,)), ((), ())),
                          preferred_element_type=jnp.float32)
    a_d = lax.dot_general(ad2_ref[...], h2t, (((1,), (0,)), ((), ())),
                          preferred_element_type=jnp.float32)
    p_self = jnp.exp(_lrelu(a_s + a_d))
    h2t_ref[...] = h2t
    as_ref[...] = a_s
    ad_ref[...] = a_d
    num0_ref[...] = h2t * p_self
    den0_ref[...] = p_self


def _tc_mid(num1, den1, b1c, W2, a2s, a2d, Sel):
    grid = (NP // BN,)
    return pl.pallas_call(
        _tc_mid_body,
        grid=grid,
        in_specs=[
            pl.BlockSpec((F1, BN), lambda i: (0, i)),
            pl.BlockSpec((HEADS, BN), lambda i: (0, i)),
            pl.BlockSpec((F1, 1), lambda i: (0, 0)),
            pl.BlockSpec((F1, F2), lambda i: (0, 0)),
            pl.BlockSpec((1, F2), lambda i: (0, 0)),
            pl.BlockSpec((1, F2), lambda i: (0, 0)),
            pl.BlockSpec((F1, HEADS), lambda i: (0, 0)),
        ],
        out_specs=[
            pl.BlockSpec((F2, BN), lambda i: (0, i)),
            pl.BlockSpec((1, BN), lambda i: (0, i)),
            pl.BlockSpec((1, BN), lambda i: (0, i)),
            pl.BlockSpec((F2, BN), lambda i: (0, i)),
            pl.BlockSpec((1, BN), lambda i: (0, i)),
        ],
        out_shape=[
            jax.ShapeDtypeStruct((F2, NP), jnp.float32),
            jax.ShapeDtypeStruct((1, NP), jnp.float32),
            jax.ShapeDtypeStruct((1, NP), jnp.float32),
            jax.ShapeDtypeStruct((F2, NP), jnp.float32),
            jax.ShapeDtypeStruct((1, NP), jnp.float32),
        ],
    )(num1, den1, b1c, W2, a2s, a2d, Sel)


def _sc_edge2(src, dst, a_src2, a_dst2, h2T, num_init, den_init):
    mesh = plsc.VectorSubcoreMesh(**_MESH)
    EH = E // 2

    @functools.partial(
        pl.kernel,
        out_type=[
            jax.ShapeDtypeStruct((2, F2, NP), jnp.float32),
            jax.ShapeDtypeStruct((2, NP), jnp.float32),
        ],
        mesh=mesh,
        scratch_types=[
            pltpu.VMEM((2, NP), jnp.float32),
            pltpu.VMEM((2, NP), jnp.float32),
            pltpu.VMEM((1, NP), jnp.float32),
            pltpu.VMEM((1, NP), jnp.float32),
            pltpu.VMEM((1, NP), jnp.float32),
            pltpu.VMEM((CH,), jnp.int32),
            pltpu.VMEM((CH,), jnp.int32),
            pltpu.VMEM((CH,), jnp.int32),
            pltpu.VMEM((CH,), jnp.int32),
            pltpu.SemaphoreType.DMA,
            pltpu.SemaphoreType.DMA,
        ],
        compiler_params=_SC_PARAMS,
    )
    def k(src_h, dst_h, as_h, ad_h, h2_h, n0_h, d0_h, num_o, den_o,
          hc, acc, asr, adr, dacc, sb0, db0, sb1, db1, sem0, sem1):
        t = lax.axis_index("s") * 2 + lax.axis_index("c")
        half = t // 16
        fp = t % 16
        ebase = half * EH
        pltpu.sync_copy(h2_h.at[pl.ds(2 * fp, 2)], hc)
        pltpu.sync_copy(as_h, asr)
        pltpu.sync_copy(ad_h, adr)

        @pl.when(half == 0)
        def _():
            pltpu.sync_copy(n0_h.at[pl.ds(2 * fp, 2)], acc)
            pltpu.sync_copy(d0_h, dacc)

        @pl.when(half == 1)
        def _():
            z = jnp.zeros((16,), jnp.float32)

            def zb(i, _):
                acc[0, pl.ds(i * 16, 16)] = z
                acc[1, pl.ds(i * 16, 16)] = z
                dacc[0, pl.ds(i * 16, 16)] = z
                return 0

            lax.fori_loop(0, NP // 16, zb, 0)

        f0 = jnp.full((16,), 0, jnp.int32)
        f1 = jnp.full((16,), 1, jnp.int32)

        def start(c, sb, db, sem):
            pltpu.async_copy(src_h.at[pl.ds(ebase + c * CH, CH)], sb, sem)
            pltpu.async_copy(dst_h.at[pl.ds(ebase + c * CH, CH)], db, sem)

        def drain(sb, db, sem):
            pltpu.make_async_copy(src_h.at[pl.ds(0, CH)], sb, sem).wait()
            pltpu.make_async_copy(dst_h.at[pl.ds(0, CH)], db, sem).wait()

        def inner(sb, db):
            @plsc.parallel_loop(0, CH // 16, unroll=8)
            def blk(i):
                o = i * 16
                sv = sb[pl.ds(o, 16)]
                dv = db[pl.ds(o, 16)]
                e = (plsc.load_gather(asr, [f0, sv])
                     + plsc.load_gather(adr, [f0, dv]))
                p = jnp.exp(jnp.maximum(e, 0.2 * e))
                plsc.addupdate_scatter(dacc, [f0, dv], p)
                for fv in (f0, f1):
                    hv = plsc.load_gather(hc, [fv, sv])
                    plsc.addupdate_scatter(acc, [fv, dv], hv * p)

        start(0, sb0, db0, sem0)

        def chunk_body(g2, _):
            c = 2 * g2
            start(c + 1, sb1, db1, sem1)
            drain(sb0, db0, sem0)
            inner(sb0, db0)

            @pl.when(c + 2 < EH // CH)
            def _():
                start(c + 2, sb0, db0, sem0)

            drain(sb1, db1, sem1)
            inner(sb1, db1)
            return 0

        lax.fori_loop(0, EH // (2 * CH), chunk_body, 0)

        pltpu.sync_copy(acc, num_o.at[half, pl.ds(2 * fp, 2)])

        @pl.when(fp == 0)
        def _():
            pltpu.sync_copy(dacc, den_o.at[pl.ds(half, 1)])

    return k(src, dst, a_src2, a_dst2, h2T, num_init, den_init)


def _tc_post_body(num_ref, den_ref, b2_ref, lw_ref, lb_ref, out_ref):
    num = num_ref[0] + num_ref[1]
    den = den_ref[pl.ds(0, 1)] + den_ref[pl.ds(1, 1)]
    out2 = num / (den + 1e-16) + b2_ref[...]
    out2 = _elu(out2)
    y = lax.dot_general(lw_ref[...], out2, (((0,), (0,)), ((), ())),
                        preferred_element_type=jnp.float32)
    out_ref[...] = y + lb_ref[...]


def _tc_post(num2, den2, b2c, lin_W, lin_bc):
    grid = (NP // BN,)
    return pl.pallas_call(
        _tc_post_body,
        grid=grid,
        in_specs=[
            pl.BlockSpec((2, F2, BN), lambda i: (0, 0, i)),
            pl.BlockSpec((2, BN), lambda i: (0, i)),
            pl.BlockSpec((F2, 1), lambda i: (0, 0)),
            pl.BlockSpec((F2, OUT), lambda i: (0, 0)),
            pl.BlockSpec((OUT, 1), lambda i: (0, 0)),
        ],
        out_specs=pl.BlockSpec((OUT, BN), lambda i: (0, i)),
        out_shape=jax.ShapeDtypeStruct((OUT, NP), jnp.float32),
    )(num2, den2, b2c, lin_W, lin_bc)


def kernel(x, edge_index, W1, att_src1, att_dst1, b1, W2, att_src2, att_dst2,
           b2, lin_W, lin_b):
    x_pad = jnp.pad(x, ((0, NP - N), (0, 0)))
    src = edge_index[0]
    dst = edge_index[1]

    head_of = np.arange(F1) // HID
    blockmask = jnp.asarray(
        (np.arange(HEADS)[:, None] == head_of[None, :]).astype(np.float32))
    A_src = blockmask * att_src1.reshape(1, F1)
    A_dst = blockmask * att_dst1.reshape(1, F1)
    Sel = blockmask.T

    h1T, a_sT, a_dT, num0, den0 = _tc_pre1(x_pad, W1, A_src, A_dst, Sel)
    num1, den1 = _sc_edge1(src, dst, a_sT, a_dT, h1T, num0, den0)

    h2T, a2sT, a2dT, num0b, den0b = _tc_mid(
        num1, den1, b1.reshape(F1, 1), W2,
        att_src2.reshape(1, F2), att_dst2.reshape(1, F2), Sel)
    num2, den2 = _sc_edge2(src, dst, a2sT, a2dT, h2T, num0b, den0b)

    yT = _tc_post(num2, den2, b2.reshape(F2, 1), lin_W,
                  lin_b.reshape(OUT, 1))
    return yT[:, :N].T
